# baseline (speedup 1.0000x reference)
import dataclasses
import functools

import jax
import jax.numpy as jnp
from jax import lax
from jax.experimental import pallas as pl
from jax.experimental.pallas import tpu as pltpu
from jax.experimental.pallas import tpu_sc as plsc

N = 10000
E = 160000
D = 256

NUM_CORES = 2
NUM_SUBCORES = 16
NW = NUM_CORES * NUM_SUBCORES
TPN = 320
TRASH = TPN
ACC_ROWS = TPN + 1
SCAN = 2000
NSCAN = E // SCAN
GROUP = 64
DRAIN_T = 256
PEND = SCAN + DRAIN_T + 32


def _mm_body(h_ref, aw_ref, ab_ref, bw_ref, bb_ref, ah_ref, bh_ref):
    x = h_ref[...]
    dn = (((1,), (1,)), ((), ()))
    ah_ref[...] = lax.dot_general(
        x, aw_ref[...], dn, preferred_element_type=jnp.float32,
        precision=lax.Precision.HIGHEST) + ab_ref[...]
    bh_ref[...] = lax.dot_general(
        x, bw_ref[...], dn, preferred_element_type=jnp.float32,
        precision=lax.Precision.HIGHEST) + bb_ref[...]


def _matmuls(h, A_w, A_b, B_w, B_b):
    BM = 1000
    grid = (N // BM,)
    out_shape = [jax.ShapeDtypeStruct((N, D), jnp.float32)] * 2
    return pl.pallas_call(
        _mm_body,
        grid=grid,
        in_specs=[
            pl.BlockSpec((BM, D), lambda i: (i, 0)),
            pl.BlockSpec((D, D), lambda i: (0, 0)),
            pl.BlockSpec((1, D), lambda i: (0, 0)),
            pl.BlockSpec((D, D), lambda i: (0, 0)),
            pl.BlockSpec((1, D), lambda i: (0, 0)),
        ],
        out_specs=[
            pl.BlockSpec((BM, D), lambda i: (i, 0)),
            pl.BlockSpec((BM, D), lambda i: (i, 0)),
        ],
        out_shape=out_shape,
    )(h, A_w, A_b.reshape(1, D), B_w, B_b.reshape(1, D))


def _sc_segment_sum(bh, src, dst):
    mesh = plsc.VectorSubcoreMesh(core_axis_name="c", subcore_axis_name="s")
    cp = pltpu.CompilerParams()
    if "needs_layout_passes" in pltpu.CompilerParams.__dataclass_fields__:
        cp = dataclasses.replace(cp, needs_layout_passes=False)

    @functools.partial(
        pl.kernel,
        out_type=jax.ShapeDtypeStruct((N, D), jnp.float32),
        mesh=mesh,
        compiler_params=cp,
        scratch_types=[
            pltpu.VMEM((SCAN,), jnp.int32),
            pltpu.VMEM((SCAN,), jnp.int32),
            pltpu.VMEM((SCAN,), jnp.int32),
            pltpu.VMEM((SCAN,), jnp.int32),
            pltpu.VMEM((PEND,), jnp.int32),
            pltpu.VMEM((PEND,), jnp.int32),
            pltpu.VMEM((GROUP, D), jnp.float32),
            pltpu.VMEM((GROUP, D), jnp.float32),
            pltpu.VMEM((ACC_ROWS, D), jnp.float32),
            pltpu.SMEM((8,), jnp.int32),
            pltpu.SemaphoreType.DMA,
            pltpu.SemaphoreType.DMA,
            pltpu.SemaphoreType.DMA,
            pltpu.SemaphoreType.DMA,
            pltpu.SemaphoreType.DMA,
            pltpu.SemaphoreType.DMA,
        ],
    )
    def k(bh_hbm, src_hbm, dst_hbm, out_hbm,
          sbuf0, dbuf0, sbuf1, dbuf1, pgidx, pldst, rows0, rows1, acc, cnt_s,
          ssem0, dsem0, ssem1, dsem1, rsem0, rsem1):
        c = lax.axis_index("c")
        s = lax.axis_index("s")
        w = c * NUM_SUBCORES + s
        lo = w * TPN

        z16 = jnp.zeros((16,), jnp.float32)
        iota = lax.iota(jnp.int32, 16)

        def start_scan(kc, sb, db, ssem, dsem):
            base = kc * SCAN
            pltpu.async_copy(src_hbm.at[pl.ds(base, SCAN)], sb, ssem)
            pltpu.async_copy(dst_hbm.at[pl.ds(base, SCAN)], db, dsem)

        def wait_scan(sb, db, ssem, dsem):
            pltpu.make_async_copy(src_hbm.at[pl.ds(0, SCAN)], sb, ssem).wait()
            pltpu.make_async_copy(dst_hbm.at[pl.ds(0, SCAN)], db, dsem).wait()

        def start_gather(goff, rbuf, rsem):
            pltpu.async_copy(bh_hbm.at[pgidx.at[pl.ds(goff, GROUP)]],
                             rbuf, rsem)

        def wait_gather(rbuf, rsem):
            pltpu.make_async_copy(bh_hbm.at[pgidx.at[pl.ds(0, GROUP)]],
                                  rbuf, rsem).wait()

        def accumulate(goff, rbuf):
            @pl.loop(0, GROUP, step=16)
            def _(i):
                ld16 = pldst[pl.ds(goff + i, 16)]
                for t in range(16):
                    ld = ld16[t]
                    vals = [rbuf[i + t, pl.ds(j * 16, 16)]
                            for j in range(D // 16)]
                    for j in range(D // 16):
                        plsc.addupdate(acc.at[ld, pl.ds(j * 16, 16)], vals[j])

        def compact(sb, db):
            lane15 = jnp.full((16, 1), 15, jnp.int32)
            gd = lax.GatherDimensionNumbers(
                offset_dims=(), collapsed_slice_dims=(0,),
                start_index_map=(0,))

            def splat_last(v):
                return lax.gather(
                    v, lane15, gd, (1,),
                    mode=lax.GatherScatterMode.PROMISE_IN_BOUNDS)

            def step(cntv, sl):
                d = db[sl]
                sv = sb[sl]
                l = d - lo
                m = (l >= 0) & (l < TPN)
                cs = plsc.cumsum(jnp.where(m, 1, 0))
                pos = cs + cntv - 1
                plsc.store_scatter(pgidx, [pos], sv, mask=m)
                plsc.store_scatter(pldst, [pos], l, mask=m)
                return cntv + splat_last(cs)

            def body(g, cntv):
                cntv = step(cntv, pl.ds(g * 32, 16))
                return step(cntv, pl.ds(g * 32 + 16, 16))

            cntv0 = jnp.full((16,), cnt_s[0], jnp.int32)
            cntv = lax.fori_loop(0, SCAN // 32, body, cntv0)
            cnt_s[0] = step(cntv, pl.ds(SCAN - 16, 16))[0]

        def drain():
            ngroups = cnt_s[0] // GROUP

            @pl.when(ngroups > 0)
            def _():
                start_gather(0, rows0, rsem0)

                @pl.when(ngroups > 1)
                def _():
                    start_gather(GROUP, rows1, rsem1)

                @pl.loop(0, ngroups, step=2)
                def _(g):
                    wait_gather(rows0, rsem0)
                    accumulate(g * GROUP, rows0)

                    @pl.when(g + 2 < ngroups)
                    def _():
                        start_gather((g + 2) * GROUP, rows0, rsem0)

                    @pl.when(g + 1 < ngroups)
                    def _():
                        wait_gather(rows1, rsem1)
                        accumulate((g + 1) * GROUP, rows1)

                        @pl.when(g + 3 < ngroups)
                        def _():
                            start_gather((g + 3) * GROUP, rows1, rsem1)

                gbase = ngroups * GROUP
                for j in range(GROUP // 16):
                    pgidx[pl.ds(j * 16, 16)] = pgidx[pl.ds(gbase + j * 16, 16)]
                    pldst[pl.ds(j * 16, 16)] = pldst[pl.ds(gbase + j * 16, 16)]
                cnt_s[0] = cnt_s[0] - gbase

        cnt_s[0] = 0
        start_scan(0, sbuf0, dbuf0, ssem0, dsem0)

        @pl.loop(0, ACC_ROWS)
        def _(r):
            for j in range(D // 16):
                acc[r, pl.ds(j * 16, 16)] = z16

        @pl.loop(0, NSCAN, step=2)
        def _(kc):
            wait_scan(sbuf0, dbuf0, ssem0, dsem0)
            start_scan(kc + 1, sbuf1, dbuf1, ssem1, dsem1)
            compact(sbuf0, dbuf0)

            @pl.when(cnt_s[0] >= DRAIN_T)
            def _():
                drain()

            wait_scan(sbuf1, dbuf1, ssem1, dsem1)

            @pl.when(kc + 2 < NSCAN)
            def _():
                start_scan(kc + 2, sbuf0, dbuf0, ssem0, dsem0)

            compact(sbuf1, dbuf1)

            @pl.when(cnt_s[0] >= DRAIN_T)
            def _():
                drain()

        drain()
        cnt = cnt_s[0]
        for j in range(GROUP // 16):
            pos = cnt + j * 16 + iota
            plsc.store_scatter(pgidx, [pos], jnp.zeros((16,), jnp.int32))
            plsc.store_scatter(pldst, [pos], jnp.full((16,), TRASH, jnp.int32))
        start_gather(0, rows0, rsem0)
        wait_gather(rows0, rsem0)
        accumulate(0, rows0)

        nreal = N - (NW - 1) * TPN
        @pl.when(w < NW - 1)
        def _():
            pltpu.sync_copy(acc.at[pl.ds(0, TPN)], out_hbm.at[pl.ds(lo, TPN)])

        @pl.when(w == NW - 1)
        def _():
            pltpu.sync_copy(acc.at[pl.ds(0, nreal)],
                            out_hbm.at[pl.ds(lo, nreal)])

    return k(bh, src, dst)


def _bn_body(ah_ref, s_ref, g_ref, b_ref, o_ref):
    x = ah_ref[...] + s_ref[...]
    mean = jnp.mean(x, axis=0, keepdims=True)
    var = jnp.mean(jnp.square(x - mean), axis=0, keepdims=True)
    inv = lax.rsqrt(var + 1e-5)
    o_ref[...] = jnp.maximum((x - mean) * inv * g_ref[...] + b_ref[...], 0.0)


def _bn_relu(ah, ssum, gamma, beta):
    return pl.pallas_call(
        _bn_body,
        out_shape=jax.ShapeDtypeStruct((N, D), jnp.float32),
    )(ah, ssum, gamma.reshape(1, D), beta.reshape(1, D))


def kernel(h, edge_index, A_w, A_b, B_w, B_b, bn_gamma, bn_beta):
    ah, bh = _matmuls(h, A_w, A_b, B_w, B_b)
    ssum = _sc_segment_sum(bh, edge_index[0], edge_index[1])
    return _bn_relu(ah, ssum, bn_gamma, bn_beta)


# speedup vs baseline: 1.0609x; 1.0609x over previous
import dataclasses
import functools

import jax
import jax.numpy as jnp
from jax import lax
from jax.experimental import pallas as pl
from jax.experimental.pallas import tpu as pltpu
from jax.experimental.pallas import tpu_sc as plsc

N = 10000
E = 160000
D = 256

NUM_CORES = 2
NUM_SUBCORES = 16
NW = NUM_CORES * NUM_SUBCORES
TPN = 320
TRASH = TPN
ACC_ROWS = TPN + 1
SCAN = 2000
NSCAN = E // SCAN
GROUP = 64
DRAIN_T = 512
PEND = SCAN + DRAIN_T + 32


def _mm_body(h_ref, aw_ref, ab_ref, bw_ref, bb_ref, ah_ref, bh_ref):
    x = h_ref[...]
    dn = (((1,), (1,)), ((), ()))
    ah_ref[...] = lax.dot_general(
        x, aw_ref[...], dn, preferred_element_type=jnp.float32,
        precision=lax.Precision.HIGHEST) + ab_ref[...]
    bh_ref[...] = lax.dot_general(
        x, bw_ref[...], dn, preferred_element_type=jnp.float32,
        precision=lax.Precision.HIGHEST) + bb_ref[...]


def _matmuls(h, A_w, A_b, B_w, B_b):
    BM = 1000
    grid = (N // BM,)
    out_shape = [jax.ShapeDtypeStruct((N, D), jnp.float32)] * 2
    return pl.pallas_call(
        _mm_body,
        grid=grid,
        in_specs=[
            pl.BlockSpec((BM, D), lambda i: (i, 0)),
            pl.BlockSpec((D, D), lambda i: (0, 0)),
            pl.BlockSpec((1, D), lambda i: (0, 0)),
            pl.BlockSpec((D, D), lambda i: (0, 0)),
            pl.BlockSpec((1, D), lambda i: (0, 0)),
        ],
        out_specs=[
            pl.BlockSpec((BM, D), lambda i: (i, 0)),
            pl.BlockSpec((BM, D), lambda i: (i, 0)),
        ],
        out_shape=out_shape,
    )(h, A_w, A_b.reshape(1, D), B_w, B_b.reshape(1, D))


def _sc_segment_sum(bh, src, dst):
    mesh = plsc.VectorSubcoreMesh(core_axis_name="c", subcore_axis_name="s")
    cp = pltpu.CompilerParams()
    if "needs_layout_passes" in pltpu.CompilerParams.__dataclass_fields__:
        cp = dataclasses.replace(cp, needs_layout_passes=False)

    @functools.partial(
        pl.kernel,
        out_type=jax.ShapeDtypeStruct((N, D), jnp.float32),
        mesh=mesh,
        compiler_params=cp,
        scratch_types=[
            pltpu.VMEM((SCAN,), jnp.int32),
            pltpu.VMEM((SCAN,), jnp.int32),
            pltpu.VMEM((SCAN,), jnp.int32),
            pltpu.VMEM((SCAN,), jnp.int32),
            pltpu.VMEM((PEND,), jnp.int32),
            pltpu.VMEM((PEND,), jnp.int32),
            pltpu.VMEM((GROUP, D), jnp.float32),
            pltpu.VMEM((GROUP, D), jnp.float32),
            pltpu.VMEM((ACC_ROWS, D), jnp.float32),
            pltpu.SMEM((8,), jnp.int32),
            pltpu.SemaphoreType.DMA,
            pltpu.SemaphoreType.DMA,
            pltpu.SemaphoreType.DMA,
            pltpu.SemaphoreType.DMA,
            pltpu.SemaphoreType.DMA,
            pltpu.SemaphoreType.DMA,
        ],
    )
    def k(bh_hbm, src_hbm, dst_hbm, out_hbm,
          sbuf0, dbuf0, sbuf1, dbuf1, pgidx, pldst, rows0, rows1, acc, cnt_s,
          ssem0, dsem0, ssem1, dsem1, rsem0, rsem1):
        c = lax.axis_index("c")
        s = lax.axis_index("s")
        w = c * NUM_SUBCORES + s
        lo = w * TPN

        z16 = jnp.zeros((16,), jnp.float32)
        iota = lax.iota(jnp.int32, 16)

        def start_scan(kc, sb, db, ssem, dsem):
            base = kc * SCAN
            pltpu.async_copy(src_hbm.at[pl.ds(base, SCAN)], sb, ssem)
            pltpu.async_copy(dst_hbm.at[pl.ds(base, SCAN)], db, dsem)

        def wait_scan(sb, db, ssem, dsem):
            pltpu.make_async_copy(src_hbm.at[pl.ds(0, SCAN)], sb, ssem).wait()
            pltpu.make_async_copy(dst_hbm.at[pl.ds(0, SCAN)], db, dsem).wait()

        def start_gather(goff, rbuf, rsem):
            pltpu.async_copy(bh_hbm.at[pgidx.at[pl.ds(goff, GROUP)]],
                             rbuf, rsem)

        def wait_gather(rbuf, rsem):
            pltpu.make_async_copy(bh_hbm.at[pgidx.at[pl.ds(0, GROUP)]],
                                  rbuf, rsem).wait()

        def accumulate(goff, rbuf):
            @pl.loop(0, GROUP, step=16)
            def _(i):
                ld16 = pldst[pl.ds(goff + i, 16)]
                for t in range(16):
                    ld = ld16[t]
                    vals = [rbuf[i + t, pl.ds(j * 16, 16)]
                            for j in range(D // 16)]
                    for j in range(D // 16):
                        plsc.addupdate(acc.at[ld, pl.ds(j * 16, 16)], vals[j])

        def compact(sb, db):
            def step(cnt, sl):
                d = db[sl]
                sv = sb[sl]
                l = d - lo
                m = (l >= 0) & (l < TPN)
                cs = plsc.cumsum(jnp.where(m, 1, 0))
                pos = cs + (cnt - 1)
                plsc.store_scatter(pgidx, [pos], sv, mask=m)
                plsc.store_scatter(pldst, [pos], l, mask=m)
                return cnt + cs[15]

            def body(g, cnt):
                cnt = step(cnt, pl.ds(g * 32, 16))
                return step(cnt, pl.ds(g * 32 + 16, 16))

            cnt = lax.fori_loop(0, SCAN // 32, body, cnt_s[0])
            cnt_s[0] = step(cnt, pl.ds(SCAN - 16, 16))

        def drain():
            ngroups = cnt_s[0] // GROUP

            @pl.when(ngroups > 0)
            def _():
                start_gather(0, rows0, rsem0)

                @pl.when(ngroups > 1)
                def _():
                    start_gather(GROUP, rows1, rsem1)

                @pl.loop(0, ngroups, step=2)
                def _(g):
                    wait_gather(rows0, rsem0)
                    accumulate(g * GROUP, rows0)

                    @pl.when(g + 2 < ngroups)
                    def _():
                        start_gather((g + 2) * GROUP, rows0, rsem0)

                    @pl.when(g + 1 < ngroups)
                    def _():
                        wait_gather(rows1, rsem1)
                        accumulate((g + 1) * GROUP, rows1)

                        @pl.when(g + 3 < ngroups)
                        def _():
                            start_gather((g + 3) * GROUP, rows1, rsem1)

                gbase = ngroups * GROUP
                for j in range(GROUP // 16):
                    pgidx[pl.ds(j * 16, 16)] = pgidx[pl.ds(gbase + j * 16, 16)]
                    pldst[pl.ds(j * 16, 16)] = pldst[pl.ds(gbase + j * 16, 16)]
                cnt_s[0] = cnt_s[0] - gbase

        cnt_s[0] = 0
        start_scan(0, sbuf0, dbuf0, ssem0, dsem0)

        @pl.loop(0, ACC_ROWS)
        def _(r):
            for j in range(D // 16):
                acc[r, pl.ds(j * 16, 16)] = z16

        @pl.loop(0, NSCAN, step=2)
        def _(kc):
            wait_scan(sbuf0, dbuf0, ssem0, dsem0)
            start_scan(kc + 1, sbuf1, dbuf1, ssem1, dsem1)
            compact(sbuf0, dbuf0)

            @pl.when(cnt_s[0] >= DRAIN_T)
            def _():
                drain()

            wait_scan(sbuf1, dbuf1, ssem1, dsem1)

            @pl.when(kc + 2 < NSCAN)
            def _():
                start_scan(kc + 2, sbuf0, dbuf0, ssem0, dsem0)

            compact(sbuf1, dbuf1)

            @pl.when(cnt_s[0] >= DRAIN_T)
            def _():
                drain()

        drain()
        cnt = cnt_s[0]
        for j in range(GROUP // 16):
            pos = cnt + j * 16 + iota
            plsc.store_scatter(pgidx, [pos], jnp.zeros((16,), jnp.int32))
            plsc.store_scatter(pldst, [pos], jnp.full((16,), TRASH, jnp.int32))
        start_gather(0, rows0, rsem0)
        wait_gather(rows0, rsem0)
        accumulate(0, rows0)

        nreal = N - (NW - 1) * TPN
        @pl.when(w < NW - 1)
        def _():
            pltpu.sync_copy(acc.at[pl.ds(0, TPN)], out_hbm.at[pl.ds(lo, TPN)])

        @pl.when(w == NW - 1)
        def _():
            pltpu.sync_copy(acc.at[pl.ds(0, nreal)],
                            out_hbm.at[pl.ds(lo, nreal)])

    return k(bh, src, dst)


def _bn_body(ah_ref, s_ref, g_ref, b_ref, o_ref):
    x = ah_ref[...] + s_ref[...]
    mean = jnp.mean(x, axis=0, keepdims=True)
    var = jnp.mean(jnp.square(x - mean), axis=0, keepdims=True)
    inv = lax.rsqrt(var + 1e-5)
    o_ref[...] = jnp.maximum((x - mean) * inv * g_ref[...] + b_ref[...], 0.0)


def _bn_relu(ah, ssum, gamma, beta):
    return pl.pallas_call(
        _bn_body,
        out_shape=jax.ShapeDtypeStruct((N, D), jnp.float32),
    )(ah, ssum, gamma.reshape(1, D), beta.reshape(1, D))


def kernel(h, edge_index, A_w, A_b, B_w, B_b, bn_gamma, bn_beta):
    ah, bh = _matmuls(h, A_w, A_b, B_w, B_b)
    ssum = _sc_segment_sum(bh, edge_index[0], edge_index[1])
    return _bn_relu(ah, ssum, bn_gamma, bn_beta)


# speedup vs baseline: 1.0614x; 1.0005x over previous
import dataclasses
import functools

import jax
import jax.numpy as jnp
from jax import lax
from jax.experimental import pallas as pl
from jax.experimental.pallas import tpu as pltpu
from jax.experimental.pallas import tpu_sc as plsc

N = 10000
E = 160000
D = 256

NUM_CORES = 2
NUM_SUBCORES = 16
NW = NUM_CORES * NUM_SUBCORES
TPN = 320
TRASH = TPN
ACC_ROWS = TPN + 1
SCAN = 2000
NSCAN = E // SCAN
GROUP = 64
DRAIN_T = 512
PEND = SCAN + DRAIN_T + 32


def _mm_body(h_ref, aw_ref, ab_ref, bw_ref, bb_ref, ah_ref, bh_ref):
    x = h_ref[...]
    dn = (((1,), (1,)), ((), ()))
    ah_ref[...] = lax.dot_general(
        x, aw_ref[...], dn, preferred_element_type=jnp.float32,
        precision=lax.Precision.HIGHEST) + ab_ref[...]
    bh_ref[...] = lax.dot_general(
        x, bw_ref[...], dn, preferred_element_type=jnp.float32,
        precision=lax.Precision.HIGHEST) + bb_ref[...]


def _matmuls(h, A_w, A_b, B_w, B_b):
    BM = 1000
    grid = (N // BM,)
    out_shape = [jax.ShapeDtypeStruct((N, D), jnp.float32)] * 2
    return pl.pallas_call(
        _mm_body,
        grid=grid,
        in_specs=[
            pl.BlockSpec((BM, D), lambda i: (i, 0)),
            pl.BlockSpec((D, D), lambda i: (0, 0)),
            pl.BlockSpec((1, D), lambda i: (0, 0)),
            pl.BlockSpec((D, D), lambda i: (0, 0)),
            pl.BlockSpec((1, D), lambda i: (0, 0)),
        ],
        out_specs=[
            pl.BlockSpec((BM, D), lambda i: (i, 0)),
            pl.BlockSpec((BM, D), lambda i: (i, 0)),
        ],
        out_shape=out_shape,
    )(h, A_w, A_b.reshape(1, D), B_w, B_b.reshape(1, D))


def _sc_segment_sum(bh, src, dst):
    mesh = plsc.VectorSubcoreMesh(core_axis_name="c", subcore_axis_name="s")
    cp = pltpu.CompilerParams()
    if "needs_layout_passes" in pltpu.CompilerParams.__dataclass_fields__:
        cp = dataclasses.replace(cp, needs_layout_passes=False)

    @functools.partial(
        pl.kernel,
        out_type=jax.ShapeDtypeStruct((N, D), jnp.float32),
        mesh=mesh,
        compiler_params=cp,
        scratch_types=[
            pltpu.VMEM((SCAN,), jnp.int32),
            pltpu.VMEM((SCAN,), jnp.int32),
            pltpu.VMEM((SCAN,), jnp.int32),
            pltpu.VMEM((SCAN,), jnp.int32),
            pltpu.VMEM((PEND,), jnp.int32),
            pltpu.VMEM((PEND,), jnp.int32),
            pltpu.VMEM((GROUP, D), jnp.float32),
            pltpu.VMEM((GROUP, D), jnp.float32),
            pltpu.VMEM((ACC_ROWS, D), jnp.float32),
            pltpu.SMEM((8,), jnp.int32),
            pltpu.SemaphoreType.DMA,
            pltpu.SemaphoreType.DMA,
            pltpu.SemaphoreType.DMA,
            pltpu.SemaphoreType.DMA,
            pltpu.SemaphoreType.DMA,
            pltpu.SemaphoreType.DMA,
        ],
    )
    def k(bh_hbm, src_hbm, dst_hbm, out_hbm,
          sbuf0, dbuf0, sbuf1, dbuf1, pgidx, pldst, rows0, rows1, acc, cnt_s,
          ssem0, dsem0, ssem1, dsem1, rsem0, rsem1):
        c = lax.axis_index("c")
        s = lax.axis_index("s")
        w = c * NUM_SUBCORES + s
        lo = w * TPN

        z16 = jnp.zeros((16,), jnp.float32)
        iota = lax.iota(jnp.int32, 16)

        def start_scan(kc, sb, db, ssem, dsem):
            base = kc * SCAN
            pltpu.async_copy(src_hbm.at[pl.ds(base, SCAN)], sb, ssem)
            pltpu.async_copy(dst_hbm.at[pl.ds(base, SCAN)], db, dsem)

        def wait_scan(sb, db, ssem, dsem):
            pltpu.make_async_copy(src_hbm.at[pl.ds(0, SCAN)], sb, ssem).wait()
            pltpu.make_async_copy(dst_hbm.at[pl.ds(0, SCAN)], db, dsem).wait()

        def start_gather(goff, rbuf, rsem):
            pltpu.async_copy(bh_hbm.at[pgidx.at[pl.ds(goff, GROUP)]],
                             rbuf, rsem)

        def wait_gather(rbuf, rsem):
            pltpu.make_async_copy(bh_hbm.at[pgidx.at[pl.ds(0, GROUP)]],
                                  rbuf, rsem).wait()

        def accumulate(goff, rbuf):
            @plsc.parallel_loop(0, GROUP, step=16)
            def _(i):
                ld16 = pldst[pl.ds(goff + i, 16)]
                for t in range(16):
                    ld = ld16[t]
                    vals = [rbuf[i + t, pl.ds(j * 16, 16)]
                            for j in range(D // 16)]
                    for j in range(D // 16):
                        plsc.addupdate(acc.at[ld, pl.ds(j * 16, 16)], vals[j])

        def compact(sb, db):
            def step(cnt, sl):
                d = db[sl]
                sv = sb[sl]
                l = d - lo
                m = (l >= 0) & (l < TPN)
                cs = plsc.cumsum(jnp.where(m, 1, 0))
                pos = cs + (cnt - 1)
                plsc.store_scatter(pgidx, [pos], sv, mask=m)
                plsc.store_scatter(pldst, [pos], l, mask=m)
                return cnt + cs[15]

            def body(g, cnt):
                for u in range(4):
                    cnt = step(cnt, pl.ds(g * 64 + u * 16, 16))
                return cnt

            cnt = lax.fori_loop(0, SCAN // 64, body, cnt_s[0])
            cnt_s[0] = step(cnt, pl.ds(SCAN - 16, 16))

        def drain():
            ngroups = cnt_s[0] // GROUP

            @pl.when(ngroups > 0)
            def _():
                start_gather(0, rows0, rsem0)

                @pl.when(ngroups > 1)
                def _():
                    start_gather(GROUP, rows1, rsem1)

                @pl.loop(0, ngroups, step=2)
                def _(g):
                    wait_gather(rows0, rsem0)
                    accumulate(g * GROUP, rows0)

                    @pl.when(g + 2 < ngroups)
                    def _():
                        start_gather((g + 2) * GROUP, rows0, rsem0)

                    @pl.when(g + 1 < ngroups)
                    def _():
                        wait_gather(rows1, rsem1)
                        accumulate((g + 1) * GROUP, rows1)

                        @pl.when(g + 3 < ngroups)
                        def _():
                            start_gather((g + 3) * GROUP, rows1, rsem1)

                gbase = ngroups * GROUP
                for j in range(GROUP // 16):
                    pgidx[pl.ds(j * 16, 16)] = pgidx[pl.ds(gbase + j * 16, 16)]
                    pldst[pl.ds(j * 16, 16)] = pldst[pl.ds(gbase + j * 16, 16)]
                cnt_s[0] = cnt_s[0] - gbase

        cnt_s[0] = 0
        start_scan(0, sbuf0, dbuf0, ssem0, dsem0)

        @pl.loop(0, ACC_ROWS)
        def _(r):
            for j in range(D // 16):
                acc[r, pl.ds(j * 16, 16)] = z16

        @pl.loop(0, NSCAN, step=2)
        def _(kc):
            wait_scan(sbuf0, dbuf0, ssem0, dsem0)
            start_scan(kc + 1, sbuf1, dbuf1, ssem1, dsem1)
            compact(sbuf0, dbuf0)

            @pl.when(cnt_s[0] >= DRAIN_T)
            def _():
                drain()

            wait_scan(sbuf1, dbuf1, ssem1, dsem1)

            @pl.when(kc + 2 < NSCAN)
            def _():
                start_scan(kc + 2, sbuf0, dbuf0, ssem0, dsem0)

            compact(sbuf1, dbuf1)

            @pl.when(cnt_s[0] >= DRAIN_T)
            def _():
                drain()

        drain()
        cnt = cnt_s[0]
        for j in range(GROUP // 16):
            pos = cnt + j * 16 + iota
            plsc.store_scatter(pgidx, [pos], jnp.zeros((16,), jnp.int32))
            plsc.store_scatter(pldst, [pos], jnp.full((16,), TRASH, jnp.int32))
        start_gather(0, rows0, rsem0)
        wait_gather(rows0, rsem0)
        accumulate(0, rows0)

        nreal = N - (NW - 1) * TPN
        @pl.when(w < NW - 1)
        def _():
            pltpu.sync_copy(acc.at[pl.ds(0, TPN)], out_hbm.at[pl.ds(lo, TPN)])

        @pl.when(w == NW - 1)
        def _():
            pltpu.sync_copy(acc.at[pl.ds(0, nreal)],
                            out_hbm.at[pl.ds(lo, nreal)])

    return k(bh, src, dst)


def _bn_body(ah_ref, s_ref, g_ref, b_ref, o_ref):
    x = ah_ref[...] + s_ref[...]
    mean = jnp.mean(x, axis=0, keepdims=True)
    var = jnp.mean(jnp.square(x - mean), axis=0, keepdims=True)
    inv = lax.rsqrt(var + 1e-5)
    o_ref[...] = jnp.maximum((x - mean) * inv * g_ref[...] + b_ref[...], 0.0)


def _bn_relu(ah, ssum, gamma, beta):
    return pl.pallas_call(
        _bn_body,
        out_shape=jax.ShapeDtypeStruct((N, D), jnp.float32),
    )(ah, ssum, gamma.reshape(1, D), beta.reshape(1, D))


def kernel(h, edge_index, A_w, A_b, B_w, B_b, bn_gamma, bn_beta):
    ah, bh = _matmuls(h, A_w, A_b, B_w, B_b)
    ssum = _sc_segment_sum(bh, edge_index[0], edge_index[1])
    return _bn_relu(ah, ssum, bn_gamma, bn_beta)
